# baseline (speedup 1.0000x reference)
import functools

import jax
import jax.numpy as jnp
from jax import lax
from jax.experimental import pallas as pl
from jax.experimental.pallas import tpu as pltpu
from jax.experimental.pallas import tpu_sc as plsc

WARP_SIZE = 32
LANES = 16


def _make_sc_stop(n: int, dep_shape, dep_dtype):
    info = plsc.get_sparse_core_info()
    nc, ns = 1, info.num_subcores
    nw = nc * ns
    tok_per_w = n // nw
    warps_per_w = tok_per_w // WARP_SIZE
    chunk = tok_per_w // LANES
    lanes_per_warp = LANES // warps_per_w
    assert tok_per_w * nw == n and warps_per_w * WARP_SIZE == tok_per_w

    mesh = plsc.VectorSubcoreMesh(
        core_axis_name="c", subcore_axis_name="s", num_cores=nc
    )

    @functools.partial(
        pl.kernel,
        mesh=mesh,
        compiler_params=pltpu.CompilerParams(
            needs_layout_passes=False, skip_device_barrier=True
        ),
        cost_estimate=pl.CostEstimate(
            flops=400_000_000, bytes_accessed=600_000_000, transcendentals=0
        ),
        out_type=jax.ShapeDtypeStruct((n,), jnp.int32),
        scratch_types=[
            pltpu.VMEM((tok_per_w,), jnp.float32),
            pltpu.VMEM((LANES,), jnp.float32),
            pltpu.VMEM((LANES,), jnp.int32),
            pltpu.VMEM((tok_per_w,), jnp.int32),
        ],
    )
    def sc_stop(probs_hbm, dep_hbm, out_hbm, probs_v, part_v, verd_v, out_v):
        del dep_hbm
        wid = lax.axis_index("s") * nc + lax.axis_index("c")
        base = wid * tok_per_w
        pltpu.sync_copy(probs_hbm.at[pl.ds(base, tok_per_w)], probs_v)

        lane = lax.iota(jnp.int32, LANES)
        lpw_shift = lanes_per_warp.bit_length() - 1
        warp_of_lane = lane >> lpw_shift
        sub_of_lane = lane & (lanes_per_warp - 1)
        idx0 = warp_of_lane * WARP_SIZE + sub_of_lane * chunk
        acc = plsc.load_gather(probs_v, [idx0])
        for i in range(1, chunk):
            acc = jnp.minimum(acc, plsc.load_gather(probs_v, [idx0 + i]))
        part_v[...] = acc

        pbase = warp_of_lane * lanes_per_warp
        t = plsc.load_gather(part_v, [pbase])
        for s in range(1, lanes_per_warp):
            t = jnp.minimum(t, plsc.load_gather(part_v, [pbase + s]))
        bits = plsc.bitcast(jnp.float32(0.5) - t, jnp.int32)
        verd_v[...] = lax.shift_right_logical(bits, jnp.broadcast_to(jnp.int32(31), (LANES,)))

        zero = lane - lane
        for jw in range(warps_per_w):
            splat = plsc.load_gather(verd_v, [zero + jw * lanes_per_warp])
            out_v[pl.ds(jw * WARP_SIZE, LANES)] = splat
            out_v[pl.ds(jw * WARP_SIZE + LANES, LANES)] = splat
        pltpu.sync_copy(out_v, out_hbm.at[pl.ds(base, tok_per_w)])

    return sc_stop


def _make_tc_copy(rows: int, cols: int, dtype, block_rows: int = 4096):
    assert rows % block_rows == 0

    def body(src_ref, dst_ref):
        dst_ref[...] = src_ref[...]

    return pl.pallas_call(
        body,
        grid=(rows // block_rows,),
        in_specs=[pl.BlockSpec((block_rows, cols), lambda i: (i, 0))],
        out_specs=pl.BlockSpec((block_rows, cols), lambda i: (i, 0)),
        out_shape=jax.ShapeDtypeStruct((rows, cols), dtype),
        compiler_params=pltpu.CompilerParams(
            dimension_semantics=("arbitrary",),
        ),
    )


def kernel(h, halt_probs):
    n = halt_probs.shape[0]
    h_out = lax.optimization_barrier(jnp.copy(h))
    stopped_i32 = _make_sc_stop(n, h.shape, h.dtype)(halt_probs, h_out)
    return (h_out, stopped_i32.astype(jnp.bool_))


# speedup vs baseline: 1.0003x; 1.0003x over previous
import functools

import jax
import jax.numpy as jnp
from jax import lax
from jax.experimental import pallas as pl
from jax.experimental.pallas import tpu as pltpu
from jax.experimental.pallas import tpu_sc as plsc

WARP_SIZE = 32
LANES = 16


def _make_sc_stop(n: int):
    info = plsc.get_sparse_core_info()
    nc, ns = 1, info.num_subcores
    nw = nc * ns
    tok_per_w = n // nw
    warps_per_w = tok_per_w // WARP_SIZE
    chunk = tok_per_w // LANES
    lanes_per_warp = LANES // warps_per_w
    assert tok_per_w * nw == n and warps_per_w * WARP_SIZE == tok_per_w

    mesh = plsc.VectorSubcoreMesh(
        core_axis_name="c", subcore_axis_name="s", num_cores=nc
    )

    @functools.partial(
        pl.kernel,
        mesh=mesh,
        compiler_params=pltpu.CompilerParams(needs_layout_passes=False),
        out_type=jax.ShapeDtypeStruct((n,), jnp.int32),
        scratch_types=[
            pltpu.VMEM((tok_per_w,), jnp.float32),
            pltpu.VMEM((LANES,), jnp.float32),
            pltpu.VMEM((LANES,), jnp.int32),
            pltpu.VMEM((tok_per_w,), jnp.int32),
        ],
    )
    def sc_stop(probs_hbm, out_hbm, probs_v, part_v, verd_v, out_v):
        wid = lax.axis_index("s") * nc + lax.axis_index("c")
        base = wid * tok_per_w
        pltpu.sync_copy(probs_hbm.at[pl.ds(base, tok_per_w)], probs_v)

        lane = lax.iota(jnp.int32, LANES)
        lpw_shift = lanes_per_warp.bit_length() - 1
        warp_of_lane = lane >> lpw_shift
        sub_of_lane = lane & (lanes_per_warp - 1)
        idx0 = warp_of_lane * WARP_SIZE + sub_of_lane * chunk
        acc = plsc.load_gather(probs_v, [idx0])
        for i in range(1, chunk):
            acc = jnp.minimum(acc, plsc.load_gather(probs_v, [idx0 + i]))
        part_v[...] = acc

        pbase = warp_of_lane * lanes_per_warp
        t = plsc.load_gather(part_v, [pbase])
        for s in range(1, lanes_per_warp):
            t = jnp.minimum(t, plsc.load_gather(part_v, [pbase + s]))
        bits = plsc.bitcast(jnp.float32(0.5) - t, jnp.int32)
        verd_v[...] = lax.shift_right_logical(
            bits, jnp.broadcast_to(jnp.int32(31), (LANES,))
        )

        zero = lane - lane
        for jw in range(warps_per_w):
            splat = plsc.load_gather(verd_v, [zero + jw * lanes_per_warp])
            out_v[pl.ds(jw * WARP_SIZE, LANES)] = splat
            out_v[pl.ds(jw * WARP_SIZE + LANES, LANES)] = splat
        pltpu.sync_copy(out_v, out_hbm.at[pl.ds(base, tok_per_w)])

    return sc_stop


def kernel(h, halt_probs):
    n = halt_probs.shape[0]
    stopped_i32 = _make_sc_stop(n)(halt_probs)
    return (h, stopped_i32.astype(jnp.bool_))
